# baseline (speedup 1.0000x reference)
import jax
import jax.numpy as jnp
from jax.experimental import pallas as pl
from jax.experimental.pallas import tpu as pltpu


def _gate_block(e_ref, o_ref):
    e = e_ref[...]
    fine = (e > 0.5).astype(jnp.int32)
    o_ref[:, 0, :] = fine ^ 1
    o_ref[:, 1, :] = fine


def kernel(entropy):
    et = jnp.transpose(entropy, (1, 2, 0)).reshape(1024, 256)
    blk = 1024
    out = pl.pallas_call(
        _gate_block,
        grid=(1024 // blk,),
        in_specs=[pl.BlockSpec((blk, 256), lambda i: (i, 0))],
        out_specs=pl.BlockSpec((blk, 2, 256), lambda i: (i, 0, 0)),
        out_shape=jax.ShapeDtypeStruct((1024, 2, 256), jnp.int32),
        compiler_params=pltpu.CompilerParams(
            dimension_semantics=("parallel",),
        ),
    )(et)
    return jnp.transpose(out.reshape(32, 32, 2, 256), (3, 0, 1, 2))
